# baseline (speedup 1.0000x reference)
import functools

import jax
import jax.numpy as jnp
from jax import lax
from jax.experimental import pallas as pl
from jax.experimental.pallas import tpu as pltpu
from jax.experimental.pallas import tpu_sc as plsc

L = 16
NC = 2
NS = 16
NW = NC * NS
K = 1024
NCH = 3
SROWS = 8
UNROLL = 8


def _pool_body(B, C, img_hbm, spx_hbm, partial_hbm, idx_v,
               i00, i01, i10, i11, i20, i21,
               a0, a1, a2, r0, r1, r2,
               s00, s01, s10, s11, s20, s21):
    _, H, W = spx_hbm.shape
    ranges_per_batch = NW // B
    rows_t = H // ranges_per_batch
    nstrip = rows_t // SROWS
    vec_per_row = W // L
    quads_per_row = vec_per_row // UNROLL

    cid = lax.axis_index("c")
    sid = lax.axis_index("s")
    wid = sid * NC + cid
    b = wid // ranges_per_batch
    r = wid % ranges_per_batch
    row0 = r * rows_t

    bufs = ((i00, i01), (i10, i11), (i20, i21))
    sems = ((s00, s01), (s10, s11), (s20, s21))
    accs = (a0, a1, a2)
    reds = (r0, r1, r2)

    neg = jnp.full((L,), -jnp.inf, jnp.float32)
    lane_off = lax.iota(jnp.int32, L) * K

    pltpu.sync_copy(spx_hbm.at[b, pl.ds(row0, rows_t)], idx_v)

    @plsc.parallel_loop(0, L * K // L, unroll=8)
    def _init(i):
        for ch in range(NCH):
            accs[ch][pl.ds(i * L, L)] = neg

    @plsc.parallel_loop(0, rows_t * vec_per_row, unroll=4)
    def _flatten(i):
        row = i >> 5
        col = (i & (vec_per_row - 1)) * L
        idx_v[row, pl.ds(col, L)] = idx_v[row, pl.ds(col, L)] + lane_off

    def _start(c0, strip, par):
        for ch in range(NCH):
            pltpu.async_copy(
                img_hbm.at[b * C + c0 + ch,
                           pl.ds(row0 + strip * SROWS, SROWS)],
                bufs[ch][par], sems[ch][par])

    def _wait(c0, strip, par):
        for ch in range(NCH):
            pltpu.make_async_copy(
                img_hbm.at[b * C + c0 + ch,
                           pl.ds(row0 + strip * SROWS, SROWS)],
                bufs[ch][par], sems[ch][par]).wait()

    def _update_strip(strip, par):
        ia = tuple(bufs[ch][par] for ch in range(NCH))

        def _ub(i, carry):
            row = i // quads_per_row
            co = (i % quads_per_row) * (UNROLL * L)
            srow = strip * SROWS + row
            idxs = []
            vals = []
            for u in range(UNROLL):
                idxs.append(idx_v[srow, pl.ds(co + u * L, L)])
            for u in range(UNROLL):
                vals.append(tuple(ia[ch][row, pl.ds(co + u * L, L)]
                                  for ch in range(NCH)))
            for u in range(UNROLL):
                g = [plsc.load_gather(accs[ch], [idxs[u]])
                     for ch in range(NCH)]
                m = [jnp.maximum(g[ch], vals[u][ch]) for ch in range(NCH)]
                for ch in range(NCH):
                    plsc.store_scatter(accs[ch], [idxs[u]], m[ch])
            return carry

        lax.fori_loop(0, SROWS * quads_per_row, _ub, 0)

    _start(0, 0, 0)

    def _group(grp, carry):
        c0 = NCH * grp

        def _strippair(j, carry2):
            st0 = 2 * j
            _wait(c0, st0, 0)
            _start(c0, st0 + 1, 1)
            _update_strip(st0, 0)
            _wait(c0, st0 + 1, 1)

            @pl.when(j + 1 < nstrip // 2)
            def _():
                _start(c0, st0 + 2, 0)

            @pl.when(jnp.logical_and(j + 1 == nstrip // 2,
                                     grp + 1 < C // NCH))
            def _():
                _start(c0 + NCH, 0, 0)

            _update_strip(st0 + 1, 1)
            return carry2

        lax.fori_loop(0, nstrip // 2, _strippair, 0)

        @plsc.parallel_loop(0, K // L, unroll=2)
        def _reduce(g):
            for ch in range(NCH):
                m = accs[ch][pl.ds(g * L, L)]
                accs[ch][pl.ds(g * L, L)] = neg
                for l in range(1, L):
                    off = l * K + g * L
                    m = jnp.maximum(m, accs[ch][pl.ds(off, L)])
                    accs[ch][pl.ds(off, L)] = neg
                reds[ch][pl.ds(g * L, L)] = m

        for ch in range(NCH):
            pltpu.sync_copy(
                reds[ch],
                partial_hbm.at[pl.ds(
                    ((b * ranges_per_batch + r) * C + c0 + ch) * K, K)])
        return carry

    lax.fori_loop(0, C // NCH, _group, 0)


def _merge_body(p_ref, o_ref):
    o_ref[...] = jnp.max(p_ref[...], axis=1)


@jax.jit
def kernel(img, spx):
    B, C, H, W = img.shape
    img3 = img.reshape(B * C, H, W)
    ranges_per_batch = NW // B
    rows_t = H // ranges_per_batch

    mesh = plsc.VectorSubcoreMesh(
        core_axis_name="c", subcore_axis_name="s", num_cores=NC,
        num_subcores=NS)
    pool = pl.kernel(
        functools.partial(_pool_body, B, C),
        out_type=jax.ShapeDtypeStruct((B * ranges_per_batch * C * K,),
                                      jnp.float32),
        mesh=mesh,
        compiler_params=pltpu.CompilerParams(
            needs_layout_passes=False, use_tc_tiling_on_sc=True),
        scratch_types=(
            [pltpu.VMEM((rows_t, W), jnp.int32)]
            + [pltpu.VMEM((SROWS, W), jnp.float32)
               for _ in range(2 * NCH)]
            + [pltpu.VMEM((L * K,), jnp.float32) for _ in range(NCH)]
            + [pltpu.VMEM((K,), jnp.float32) for _ in range(NCH)]
            + [pltpu.SemaphoreType.DMA for _ in range(2 * NCH)]
        ),
    )
    partial = pool(img3, spx).reshape(B, ranges_per_batch, C, K)

    out = pl.pallas_call(
        _merge_body,
        grid=(B,),
        in_specs=[pl.BlockSpec((1, ranges_per_batch, C, K),
                               lambda i: (i, 0, 0, 0))],
        out_specs=pl.BlockSpec((1, C, K), lambda i: (i, 0, 0)),
        out_shape=jax.ShapeDtypeStruct((B, C, K), jnp.float32),
    )(partial)
    return out


# speedup vs baseline: 1.0371x; 1.0371x over previous
import functools

import jax
import jax.numpy as jnp
from jax import lax
from jax.experimental import pallas as pl
from jax.experimental.pallas import tpu as pltpu
from jax.experimental.pallas import tpu_sc as plsc

L = 16
NC = 2
NS = 16
NW = NC * NS
K = 1024
NREP = 17
NCH = 3
SROWS = 8
UNROLL = 4


def _pool_body(B, C, img_hbm, spx_hbm, partial_hbm, idx_v,
               i00, i01, i10, i11, i20, i21,
               a0, a1, a2, r0, r1, r2,
               s00, s01, s10, s11, s20, s21):
    _, H, W = spx_hbm.shape
    ranges_per_batch = NW // B
    rows_t = H // ranges_per_batch
    nstrip = rows_t // SROWS
    vec_per_row = W // L
    quads_per_row = vec_per_row // UNROLL

    cid = lax.axis_index("c")
    sid = lax.axis_index("s")
    wid = sid * NC + cid
    b = wid // ranges_per_batch
    r = wid % ranges_per_batch
    row0 = r * rows_t

    bufs = ((i00, i01), (i10, i11), (i20, i21))
    sems = ((s00, s01), (s10, s11), (s20, s21))
    accs = (a0, a1, a2)
    reds = (r0, r1, r2)

    neg = jnp.full((L,), -jnp.inf, jnp.float32)

    pltpu.sync_copy(spx_hbm.at[b, pl.ds(row0, rows_t)], idx_v)

    @plsc.parallel_loop(0, NREP * K // L, unroll=8)
    def _init(i):
        for ch in range(NCH):
            accs[ch][pl.ds(i * L, L)] = neg

    def _flatten(i, rmax):
        row = i >> 5
        col = (i & (vec_per_row - 1)) * L
        lab = idx_v[row, pl.ds(col, L)]
        cnt, _ = plsc.scan_count(lab)
        idx_v[row, pl.ds(col, L)] = cnt * K + lab
        return jnp.maximum(rmax, cnt)

    rmax = lax.fori_loop(0, rows_t * vec_per_row, _flatten,
                         jnp.zeros((L,), jnp.int32))
    n_rep = jnp.max(rmax) + 1

    def _start(c0, strip, par):
        for ch in range(NCH):
            pltpu.async_copy(
                img_hbm.at[b * C + c0 + ch,
                           pl.ds(row0 + strip * SROWS, SROWS)],
                bufs[ch][par], sems[ch][par])

    def _wait(c0, strip, par):
        for ch in range(NCH):
            pltpu.make_async_copy(
                img_hbm.at[b * C + c0 + ch,
                           pl.ds(row0 + strip * SROWS, SROWS)],
                bufs[ch][par], sems[ch][par]).wait()

    def _update_strip(strip, par):
        ia = tuple(bufs[ch][par] for ch in range(NCH))

        def _ub(i, carry):
            row = i // quads_per_row
            co = (i % quads_per_row) * (UNROLL * L)
            srow = strip * SROWS + row
            idxs = []
            vals = []
            for u in range(UNROLL):
                idxs.append(idx_v[srow, pl.ds(co + u * L, L)])
            for u in range(UNROLL):
                vals.append(tuple(ia[ch][row, pl.ds(co + u * L, L)]
                                  for ch in range(NCH)))
            for u in range(UNROLL):
                g = [plsc.load_gather(accs[ch], [idxs[u]])
                     for ch in range(NCH)]
                m = [jnp.maximum(g[ch], vals[u][ch]) for ch in range(NCH)]
                for ch in range(NCH):
                    plsc.store_scatter(accs[ch], [idxs[u]], m[ch])
            return carry

        lax.fori_loop(0, SROWS * quads_per_row, _ub, 0)

    _start(0, 0, 0)

    def _group(grp, carry):
        c0 = NCH * grp

        def _strippair(j, carry2):
            st0 = 2 * j
            _wait(c0, st0, 0)
            _start(c0, st0 + 1, 1)
            _update_strip(st0, 0)
            _wait(c0, st0 + 1, 1)

            @pl.when(j + 1 < nstrip // 2)
            def _():
                _start(c0, st0 + 2, 0)

            @pl.when(jnp.logical_and(j + 1 == nstrip // 2,
                                     grp + 1 < C // NCH))
            def _():
                _start(c0 + NCH, 0, 0)

            _update_strip(st0 + 1, 1)
            return carry2

        lax.fori_loop(0, nstrip // 2, _strippair, 0)

        @plsc.parallel_loop(0, K // L, unroll=4)
        def _red0(g):
            for ch in range(NCH):
                reds[ch][pl.ds(g * L, L)] = accs[ch][pl.ds(g * L, L)]
                accs[ch][pl.ds(g * L, L)] = neg

        def _redrep(rep, carry2):
            @plsc.parallel_loop(0, K // L, unroll=4)
            def _redr(g):
                for ch in range(NCH):
                    off = rep * K + g * L
                    v = accs[ch][pl.ds(off, L)]
                    accs[ch][pl.ds(off, L)] = neg
                    reds[ch][pl.ds(g * L, L)] = jnp.maximum(
                        reds[ch][pl.ds(g * L, L)], v)
            return carry2

        lax.fori_loop(1, n_rep, _redrep, 0)

        for ch in range(NCH):
            pltpu.sync_copy(
                reds[ch],
                partial_hbm.at[pl.ds(
                    ((b * ranges_per_batch + r) * C + c0 + ch) * K, K)])
        return carry

    lax.fori_loop(0, C // NCH, _group, 0)


def _merge_body(p_ref, o_ref):
    o_ref[...] = jnp.max(p_ref[...], axis=1)


@jax.jit
def kernel(img, spx):
    B, C, H, W = img.shape
    img3 = img.reshape(B * C, H, W)
    ranges_per_batch = NW // B
    rows_t = H // ranges_per_batch

    mesh = plsc.VectorSubcoreMesh(
        core_axis_name="c", subcore_axis_name="s", num_cores=NC,
        num_subcores=NS)
    pool = pl.kernel(
        functools.partial(_pool_body, B, C),
        out_type=jax.ShapeDtypeStruct((B * ranges_per_batch * C * K,),
                                      jnp.float32),
        mesh=mesh,
        compiler_params=pltpu.CompilerParams(
            needs_layout_passes=False, use_tc_tiling_on_sc=True),
        scratch_types=(
            [pltpu.VMEM((rows_t, W), jnp.int32)]
            + [pltpu.VMEM((SROWS, W), jnp.float32)
               for _ in range(2 * NCH)]
            + [pltpu.VMEM((NREP * K,), jnp.float32) for _ in range(NCH)]
            + [pltpu.VMEM((K,), jnp.float32) for _ in range(NCH)]
            + [pltpu.SemaphoreType.DMA for _ in range(2 * NCH)]
        ),
    )
    partial = pool(img3, spx).reshape(B, ranges_per_batch, C, K)

    out = pl.pallas_call(
        _merge_body,
        grid=(B,),
        in_specs=[pl.BlockSpec((1, ranges_per_batch, C, K),
                               lambda i: (i, 0, 0, 0))],
        out_specs=pl.BlockSpec((1, C, K), lambda i: (i, 0, 0)),
        out_shape=jax.ShapeDtypeStruct((B, C, K), jnp.float32),
    )(partial)
    return out
